# baseline (speedup 1.0000x reference)
import jax
import jax.numpy as jnp
from jax.experimental import pallas as pl
from jax.experimental.pallas import tpu as pltpu

F32 = jnp.float32
BF16 = jnp.bfloat16


def _conv_pass(x, wm, scale, shift, bias, *, post_sigmoid, want_stats, ht,
               out_cfirst=False, in_nchw=False):
    if in_nchw:
        B, Cin, H, W = x.shape
    else:
        B, H, W, Cin = x.shape
    WP = W + 8
    Cout = wm.shape[1] // 3
    nt = H // ht
    assert nt * ht == H
    pre_bn = scale is not None

    def body(*refs):
        it = iter(refs)
        x_ref = next(it)
        top_ref = next(it)
        bot_ref = next(it)
        wm_ref = next(it)
        sc_ref = next(it) if pre_bn else None
        sh_ref = next(it) if pre_bn else None
        b_ref = next(it) if bias is not None else None
        y_ref = next(it)
        st_ref = next(it) if want_stats else None
        xp_ref = next(it)

        b = pl.program_id(0)
        t = pl.program_id(1)

        if want_stats:
            @pl.when(jnp.logical_and(b == 0, t == 0))
            def _init():
                st_ref[...] = jnp.zeros_like(st_ref)

        xp_ref[:, W:WP, :] = jnp.zeros_like(xp_ref[:, W:WP, :])

        def prep(v):
            if in_nchw:
                v = jnp.transpose(v.astype(BF16), (1, 2, 0))
            if pre_bn:
                v = (v.astype(BF16) * sc_ref[0].astype(BF16)
                     + sh_ref[0].astype(BF16))
                v = jnp.maximum(v, jnp.zeros((), BF16))
            return v.astype(BF16)

        xp_ref[1:ht + 1, 0:W, :] = prep(x_ref[0])

        @pl.when(t == 0)
        def _():
            xp_ref[0:1, :, :] = jnp.zeros_like(xp_ref[0:1, :, :])

        @pl.when(t > 0)
        def _():
            tv = top_ref[0][:, 7:8, :] if in_nchw else top_ref[0]
            xp_ref[0:1, 0:W, :] = prep(tv)

        @pl.when(t == nt - 1)
        def _():
            xp_ref[ht + 1:ht + 2, :, :] = jnp.zeros_like(xp_ref[0:1, :, :])

        @pl.when(t < nt - 1)
        def _():
            bv = bot_ref[0][:, 0:1, :] if in_nchw else bot_ref[0]
            xp_ref[ht + 1:ht + 2, 0:W, :] = prep(bv)

        dn = (((1,), (0,)), ((), ()))
        xs = jnp.concatenate(
            [xp_ref[0:ht], xp_ref[1:ht + 1], xp_ref[2:ht + 2]], axis=-1)
        z = jax.lax.dot_general(
            xs.reshape(ht * WP, 3 * Cin), wm_ref[...],
            dimension_numbers=dn,
            preferred_element_type=F32).astype(BF16).reshape(ht, WP, 3 * Cout)
        yt = (jnp.concatenate([z[:, WP - 1:WP, 0:Cout],
                               z[:, 0:W - 1, 0:Cout]], axis=1)
              + z[:, 0:W, Cout:2 * Cout]
              + z[:, 1:W + 1, 2 * Cout:3 * Cout])
        if bias is not None:
            yt = yt + b_ref[0].astype(BF16)
        if post_sigmoid:
            yt = jax.nn.sigmoid(yt)
        yb = yt.astype(BF16)
        if out_cfirst:
            y_ref[0] = jnp.transpose(yb.reshape(ht * W, Cout), (1, 0))
        else:
            y_ref[0] = yb
        if want_stats:
            yf = yb.reshape(ht * W, Cout)
            pair = jnp.concatenate([yf, yf * yf], axis=-1)
            ones = jnp.ones((1, ht * W), BF16)
            st = jax.lax.dot_general(
                ones, pair, dimension_numbers=(((1,), (0,)), ((), ())),
                preferred_element_type=F32)
            st_ref[0:1, :] += st[:, 0:Cout]
            st_ref[1:2, :] += st[:, Cout:2 * Cout]

    if in_nchw:
        hb = ht // 8
        in_specs = [
            pl.BlockSpec((1, Cin, ht, W), lambda b, t: (b, 0, t, 0)),
            pl.BlockSpec((1, Cin, 8, W),
                         lambda b, t: (b, 0, jnp.maximum(t * hb - 1, 0), 0)),
            pl.BlockSpec((1, Cin, 8, W),
                         lambda b, t: (b, 0, jnp.minimum((t + 1) * hb, H // 8 - 1), 0)),
        ]
    else:
        in_specs = [
            pl.BlockSpec((1, ht, W, Cin), lambda b, t: (b, t, 0, 0)),
            pl.BlockSpec((1, 1, W, Cin),
                         lambda b, t: (b, jnp.maximum(t * ht - 1, 0), 0, 0)),
            pl.BlockSpec((1, 1, W, Cin),
                         lambda b, t: (b, jnp.minimum((t + 1) * ht, H - 1), 0, 0)),
        ]
    in_specs.append(pl.BlockSpec((3 * Cin, 3 * Cout), lambda b, t: (0, 0)))
    inputs = [x, x, x, wm]
    if pre_bn:
        in_specs += [pl.BlockSpec((1, Cin), lambda b, t: (0, 0))] * 2
        inputs += [scale, shift]
    if bias is not None:
        in_specs.append(pl.BlockSpec((1, Cout), lambda b, t: (0, 0)))
        inputs.append(bias)

    if out_cfirst:
        out_shape = [jax.ShapeDtypeStruct((B, Cout, H * W), BF16)]
        out_specs = [pl.BlockSpec((1, Cout, ht * W), lambda b, t: (b, 0, t))]
    else:
        out_shape = [jax.ShapeDtypeStruct((B, H, W, Cout), BF16)]
        out_specs = [pl.BlockSpec((1, ht, W, Cout), lambda b, t: (b, t, 0, 0))]
    if want_stats:
        out_shape.append(jax.ShapeDtypeStruct((2, Cout), F32))
        out_specs.append(pl.BlockSpec((2, Cout), lambda b, t: (0, 0)))

    res = pl.pallas_call(
        body,
        grid=(B, nt),
        in_specs=in_specs,
        out_specs=out_specs,
        out_shape=out_shape,
        scratch_shapes=[pltpu.VMEM((ht + 2, WP, Cin), BF16)],
    )(*inputs)
    return res if want_stats else res[0]


def _mlp_pass(xt, w1, fb1, w2, fb2, lng, lnb, w3, fb3, *, kc):
    M, K = xt.shape
    NH = w1.shape[0]
    E = w3.shape[0]
    nk = K // kc
    assert nk * kc == K
    dn_t = (((1,), (1,)), ((), ()))

    def body(x_ref, w1_ref, fb1_ref, w2_ref, fb2_ref, g_ref, b_ref,
             w3_ref, fb3_ref, o_ref, acc_ref):
        k = pl.program_id(0)

        @pl.when(k == 0)
        def _init():
            acc_ref[...] = jnp.zeros_like(acc_ref)

        acc_ref[...] += jax.lax.dot_general(
            x_ref[...], w1_ref[...],
            dimension_numbers=dn_t, preferred_element_type=F32)

        @pl.when(k == nk - 1)
        def _head():
            h1 = jnp.maximum(acc_ref[...] + fb1_ref[0], 0.0)
            z2 = jax.lax.dot_general(
                h1.astype(BF16), w2_ref[...],
                dimension_numbers=dn_t, preferred_element_type=F32) + fb2_ref[0]
            m = jnp.mean(z2, axis=-1, keepdims=True)
            v = jnp.mean(z2 * z2, axis=-1, keepdims=True) - m * m
            h2 = (z2 - m) * jax.lax.rsqrt(v + 1e-5) * g_ref[0] + b_ref[0]
            h2 = jnp.maximum(h2, 0.0)
            o_ref[...] = jax.lax.dot_general(
                h2.astype(BF16), w3_ref[...],
                dimension_numbers=dn_t, preferred_element_type=F32) + fb3_ref[0]

    return pl.pallas_call(
        body,
        grid=(nk,),
        in_specs=[
            pl.BlockSpec((M, kc), lambda k: (0, k)),
            pl.BlockSpec((NH, kc), lambda k: (0, k)),
            pl.BlockSpec((1, NH), lambda k: (0, 0)),
            pl.BlockSpec((NH, NH), lambda k: (0, 0)),
            pl.BlockSpec((1, NH), lambda k: (0, 0)),
            pl.BlockSpec((1, NH), lambda k: (0, 0)),
            pl.BlockSpec((1, NH), lambda k: (0, 0)),
            pl.BlockSpec((E, NH), lambda k: (0, 0)),
            pl.BlockSpec((1, E), lambda k: (0, 0)),
        ],
        out_specs=pl.BlockSpec((M, E), lambda k: (0, 0)),
        out_shape=jax.ShapeDtypeStruct((M, E), F32),
        scratch_shapes=[pltpu.VMEM((M, NH), F32)],
    )(xt, w1, fb1, w2, fb2, lng, lnb, w3, fb3)


def _wmat(w):
    o, i, _, _ = w.shape
    return jnp.transpose(w, (2, 1, 3, 0)).reshape(3 * i, 3 * o).astype(BF16)


def _bn_fold(st, g, be, n):
    mean = st[0] / n
    var = st[1] / n - mean * mean
    sc = g / jnp.sqrt(var + 1e-5)
    sh = be - mean * sc
    return sc[None, :], sh[None, :]


def _best_div(n, cap):
    return max(d for d in range(1, cap + 1) if n % d == 0)


def kernel(obs, w1, b1, g1, be1, w2, b2, g2, be2, w3, b3, g3, be3, w4, b4,
           fw1, fb1, fw2, fb2, lng, lnb, fw3, fb3):
    B, C, H, W = obs.shape
    K = w4.shape[0]
    n = float(B * H * W)
    ht = 56 if H % 56 == 0 else (16 if H % 16 == 0 else H)
    ht1 = 56 if H % 56 == 0 else (32 if H % 32 == 0 else H)

    y1, st1 = _conv_pass(obs, _wmat(w1), None, None, None,
                         post_sigmoid=False, want_stats=True, ht=ht1,
                         in_nchw=True)
    sc1, sh1 = _bn_fold(st1, g1, be1, n)
    y2, st2 = _conv_pass(y1, _wmat(w2), sc1, sh1, None,
                         post_sigmoid=False, want_stats=True, ht=ht)
    sc2, sh2 = _bn_fold(st2, g2, be2, n)
    y3, st3 = _conv_pass(y2, _wmat(w3), sc2, sh2, None,
                         post_sigmoid=False, want_stats=True, ht=ht)
    sc3, sh3 = _bn_fold(st3, g3, be3, n)
    fm = _conv_pass(y3, _wmat(w4), sc3, sh3, b4[None, :].astype(F32),
                    post_sigmoid=True, want_stats=False, ht=ht,
                    out_cfirst=True)

    xt = fm.reshape(B * K, H * W)
    out = _mlp_pass(xt, fw1.astype(BF16), fb1[None, :],
                    fw2.astype(BF16), fb2[None, :],
                    lng[None, :], lnb[None, :],
                    fw3.astype(BF16), fb3[None, :], kc=_best_div(H * W, 4096))
    return out.reshape(B, K, -1)


# speedup vs baseline: 1.1620x; 1.1620x over previous
import jax
import jax.numpy as jnp
from jax.experimental import pallas as pl
from jax.experimental.pallas import tpu as pltpu

F32 = jnp.float32
BF16 = jnp.bfloat16


def _conv_pass(x, wm, scale, shift, bias, *, post_sigmoid, want_stats, ht,
               out_cfirst=False, in_nchw=False):
    if in_nchw:
        B, Cin, H, W = x.shape
    else:
        B, H, W, Cin = x.shape
    WP = W + 8
    Cout = wm.shape[1] // 3
    nt = H // ht
    assert nt * ht == H
    pre_bn = scale is not None

    def body(*refs):
        it = iter(refs)
        x_ref = next(it)
        top_ref = next(it)
        bot_ref = next(it)
        wm_ref = next(it)
        sc_ref = next(it) if pre_bn else None
        sh_ref = next(it) if pre_bn else None
        b_ref = next(it) if bias is not None else None
        y_ref = next(it)
        st_ref = next(it) if want_stats else None
        xp_ref = next(it)

        b = pl.program_id(0)
        t = pl.program_id(1)

        if want_stats:
            @pl.when(jnp.logical_and(b == 0, t == 0))
            def _init():
                st_ref[...] = jnp.zeros_like(st_ref)

        xp_ref[:, W:WP, :] = jnp.zeros_like(xp_ref[:, W:WP, :])

        def prep(v):
            if in_nchw:
                v = jnp.transpose(v.astype(BF16), (1, 2, 0))
            if pre_bn:
                v = (v.astype(BF16) * sc_ref[0].astype(BF16)
                     + sh_ref[0].astype(BF16))
                v = jnp.maximum(v, jnp.zeros((), BF16))
            return v.astype(BF16)

        xp_ref[1:ht + 1, 0:W, :] = prep(x_ref[0])

        @pl.when(t == 0)
        def _():
            xp_ref[0:1, :, :] = jnp.zeros_like(xp_ref[0:1, :, :])

        @pl.when(t > 0)
        def _():
            tv = top_ref[0][:, 7:8, :] if in_nchw else top_ref[0]
            xp_ref[0:1, 0:W, :] = prep(tv)

        @pl.when(t == nt - 1)
        def _():
            xp_ref[ht + 1:ht + 2, :, :] = jnp.zeros_like(xp_ref[0:1, :, :])

        @pl.when(t < nt - 1)
        def _():
            bv = bot_ref[0][:, 0:1, :] if in_nchw else bot_ref[0]
            xp_ref[ht + 1:ht + 2, 0:W, :] = prep(bv)

        dn = (((1,), (0,)), ((), ()))
        z = sum(
            jax.lax.dot_general(
                xp_ref[dy:dy + ht].reshape(ht * WP, Cin),
                wm_ref[Cin * dy:Cin * (dy + 1), :],
                dimension_numbers=dn, preferred_element_type=F32)
            for dy in range(3)).astype(BF16).reshape(ht, WP, 3 * Cout)
        yt = (jnp.concatenate([z[:, WP - 1:WP, 0:Cout],
                               z[:, 0:W - 1, 0:Cout]], axis=1)
              + z[:, 0:W, Cout:2 * Cout]
              + z[:, 1:W + 1, 2 * Cout:3 * Cout])
        if bias is not None:
            yt = yt + b_ref[0].astype(BF16)
        if post_sigmoid:
            yt = jax.nn.sigmoid(yt)
        yb = yt.astype(BF16)
        if out_cfirst:
            y_ref[0] = jnp.transpose(yb.reshape(ht * W, Cout), (1, 0))
        else:
            y_ref[0] = yb
        if want_stats:
            yf = yb.reshape(ht * W, Cout)
            pair = jnp.concatenate([yf, yf * yf], axis=-1)
            ones = jnp.ones((1, ht * W), BF16)
            st = jax.lax.dot_general(
                ones, pair, dimension_numbers=(((1,), (0,)), ((), ())),
                preferred_element_type=F32)
            st_ref[0:1, :] += st[:, 0:Cout]
            st_ref[1:2, :] += st[:, Cout:2 * Cout]

    if in_nchw:
        hb = ht // 8
        in_specs = [
            pl.BlockSpec((1, Cin, ht, W), lambda b, t: (b, 0, t, 0)),
            pl.BlockSpec((1, Cin, 8, W),
                         lambda b, t: (b, 0, jnp.maximum(t * hb - 1, 0), 0)),
            pl.BlockSpec((1, Cin, 8, W),
                         lambda b, t: (b, 0, jnp.minimum((t + 1) * hb, H // 8 - 1), 0)),
        ]
    else:
        in_specs = [
            pl.BlockSpec((1, ht, W, Cin), lambda b, t: (b, t, 0, 0)),
            pl.BlockSpec((1, 1, W, Cin),
                         lambda b, t: (b, jnp.maximum(t * ht - 1, 0), 0, 0)),
            pl.BlockSpec((1, 1, W, Cin),
                         lambda b, t: (b, jnp.minimum((t + 1) * ht, H - 1), 0, 0)),
        ]
    in_specs.append(pl.BlockSpec((3 * Cin, 3 * Cout), lambda b, t: (0, 0)))
    inputs = [x, x, x, wm]
    if pre_bn:
        in_specs += [pl.BlockSpec((1, Cin), lambda b, t: (0, 0))] * 2
        inputs += [scale, shift]
    if bias is not None:
        in_specs.append(pl.BlockSpec((1, Cout), lambda b, t: (0, 0)))
        inputs.append(bias)

    if out_cfirst:
        out_shape = [jax.ShapeDtypeStruct((B, Cout, H * W), BF16)]
        out_specs = [pl.BlockSpec((1, Cout, ht * W), lambda b, t: (b, 0, t))]
    else:
        out_shape = [jax.ShapeDtypeStruct((B, H, W, Cout), BF16)]
        out_specs = [pl.BlockSpec((1, ht, W, Cout), lambda b, t: (b, t, 0, 0))]
    if want_stats:
        out_shape.append(jax.ShapeDtypeStruct((2, Cout), F32))
        out_specs.append(pl.BlockSpec((2, Cout), lambda b, t: (0, 0)))

    res = pl.pallas_call(
        body,
        grid=(B, nt),
        in_specs=in_specs,
        out_specs=out_specs,
        out_shape=out_shape,
        scratch_shapes=[pltpu.VMEM((ht + 2, WP, Cin), BF16)],
    )(*inputs)
    return res if want_stats else res[0]


def _mlp_pass(xt, w1, fb1, w2, fb2, lng, lnb, w3, fb3, *, kc):
    M, K = xt.shape
    NH = w1.shape[0]
    E = w3.shape[0]
    nk = K // kc
    assert nk * kc == K
    dn_t = (((1,), (1,)), ((), ()))

    def body(x_ref, w1_ref, fb1_ref, w2_ref, fb2_ref, g_ref, b_ref,
             w3_ref, fb3_ref, o_ref, acc_ref):
        k = pl.program_id(0)

        @pl.when(k == 0)
        def _init():
            acc_ref[...] = jnp.zeros_like(acc_ref)

        acc_ref[...] += jax.lax.dot_general(
            x_ref[...], w1_ref[...],
            dimension_numbers=dn_t, preferred_element_type=F32)

        @pl.when(k == nk - 1)
        def _head():
            h1 = jnp.maximum(acc_ref[...] + fb1_ref[0], 0.0)
            z2 = jax.lax.dot_general(
                h1.astype(BF16), w2_ref[...],
                dimension_numbers=dn_t, preferred_element_type=F32) + fb2_ref[0]
            m = jnp.mean(z2, axis=-1, keepdims=True)
            v = jnp.mean(z2 * z2, axis=-1, keepdims=True) - m * m
            h2 = (z2 - m) * jax.lax.rsqrt(v + 1e-5) * g_ref[0] + b_ref[0]
            h2 = jnp.maximum(h2, 0.0)
            o_ref[...] = jax.lax.dot_general(
                h2.astype(BF16), w3_ref[...],
                dimension_numbers=dn_t, preferred_element_type=F32) + fb3_ref[0]

    return pl.pallas_call(
        body,
        grid=(nk,),
        in_specs=[
            pl.BlockSpec((M, kc), lambda k: (0, k)),
            pl.BlockSpec((NH, kc), lambda k: (0, k)),
            pl.BlockSpec((1, NH), lambda k: (0, 0)),
            pl.BlockSpec((NH, NH), lambda k: (0, 0)),
            pl.BlockSpec((1, NH), lambda k: (0, 0)),
            pl.BlockSpec((1, NH), lambda k: (0, 0)),
            pl.BlockSpec((1, NH), lambda k: (0, 0)),
            pl.BlockSpec((E, NH), lambda k: (0, 0)),
            pl.BlockSpec((1, E), lambda k: (0, 0)),
        ],
        out_specs=pl.BlockSpec((M, E), lambda k: (0, 0)),
        out_shape=jax.ShapeDtypeStruct((M, E), F32),
        scratch_shapes=[pltpu.VMEM((M, NH), F32)],
    )(xt, w1, fb1, w2, fb2, lng, lnb, w3, fb3)


def _wmat(w):
    o, i, _, _ = w.shape
    return jnp.transpose(w, (2, 1, 3, 0)).reshape(3 * i, 3 * o).astype(BF16)


def _bn_fold(st, g, be, n):
    mean = st[0] / n
    var = st[1] / n - mean * mean
    sc = g / jnp.sqrt(var + 1e-5)
    sh = be - mean * sc
    return sc[None, :], sh[None, :]


def _best_div(n, cap):
    return max(d for d in range(1, cap + 1) if n % d == 0)


def kernel(obs, w1, b1, g1, be1, w2, b2, g2, be2, w3, b3, g3, be3, w4, b4,
           fw1, fb1, fw2, fb2, lng, lnb, fw3, fb3):
    B, C, H, W = obs.shape
    K = w4.shape[0]
    n = float(B * H * W)
    ht = 112 if H % 112 == 0 else (16 if H % 16 == 0 else H)
    ht1 = 112 if H % 112 == 0 else (32 if H % 32 == 0 else H)

    y1, st1 = _conv_pass(obs, _wmat(w1), None, None, None,
                         post_sigmoid=False, want_stats=True, ht=ht1,
                         in_nchw=True)
    sc1, sh1 = _bn_fold(st1, g1, be1, n)
    y2, st2 = _conv_pass(y1, _wmat(w2), sc1, sh1, None,
                         post_sigmoid=False, want_stats=True, ht=ht)
    sc2, sh2 = _bn_fold(st2, g2, be2, n)
    y3, st3 = _conv_pass(y2, _wmat(w3), sc2, sh2, None,
                         post_sigmoid=False, want_stats=True, ht=ht)
    sc3, sh3 = _bn_fold(st3, g3, be3, n)
    fm = _conv_pass(y3, _wmat(w4), sc3, sh3, b4[None, :].astype(F32),
                    post_sigmoid=True, want_stats=False, ht=ht,
                    out_cfirst=True)

    xt = fm.reshape(B * K, H * W)
    out = _mlp_pass(xt, fw1.astype(BF16), fb1[None, :],
                    fw2.astype(BF16), fb2[None, :],
                    lng[None, :], lnb[None, :],
                    fw3.astype(BF16), fb3[None, :], kc=_best_div(H * W, 4096))
    return out.reshape(B, K, -1)
